# Initial kernel scaffold; baseline (speedup 1.0000x reference)
import jax
import jax.numpy as jnp
from jax.experimental import pallas as pl


def kernel(var_node_features, con_node_features, node_types, assoc_var, assoc_con, edge_index, edge_types, edge_features, var_W1, var_b1, var_W2, var_b2, con_W1, con_b1, con_W2, con_b2, c1_basis, c1_att, c1_root, c1_bias, c2_basis, c2_att, c2_root, c2_bias, c3_basis, c3_att, c3_root, c3_bias, fc1_W, fc1_b, fc4_W, fc4_b):
    raise NotImplementedError("write your pallas kernel here")


# speedup vs baseline: 7.8811x; 7.8811x over previous
import functools

import jax
import jax.numpy as jnp
from jax import lax
from jax.experimental import pallas as pl
from jax.experimental.pallas import tpu as pltpu
from jax.experimental.pallas import tpu_sc as plsc

N = 50000
E = 800000
D = 32
H = 16
NT = 16
NC = 2
G = 128

EP = 802816
LG = EP // NT // G
CG = EP // (NT * NC) // G

SROWS = 100352
SPT = SROWS // NT
TRASH = 2 * N

CROWS = 50048
CPT = CROWS // NT
CTRASH = N

AP = 28672
AG = AP // (NT * NC) // G
APW = AP // (NT * NC)

_MESH = plsc.VectorSubcoreMesh(core_axis_name="c", subcore_axis_name="s")


CHK = 4
NCH = LG // CHK


def _sc_layer_body(x_lo, x_hi, src_g, cid_g, zeros, out_lo, out_hi,
                   idx_s, idx_c, r0, r1, r2, r3, sem_i, sem_g, acc):
    c = lax.axis_index("c")
    s = lax.axis_index("s")
    rows = (r0, r1, r2, r3)
    pltpu.sync_copy(zeros, acc.at[pl.ds(s * SPT, SPT)])
    plsc.subcore_barrier()

    def work(table, out):
        def step(t, carry):
            g0 = s * LG + t * CHK
            ci = pltpu.make_async_copy(src_g.at[pl.ds(g0, CHK)], idx_s, sem_i)
            cj = pltpu.make_async_copy(cid_g.at[pl.ds(g0, CHK)], idx_c, sem_i)
            ci.start()
            cj.start()
            ci.wait()
            cj.wait()
            gs = [pltpu.make_async_copy(table.at[idx_s.at[j]], rows[j], sem_g)
                  for j in range(CHK)]
            for g in gs:
                g.start()
            for j, g in enumerate(gs):
                g.wait()
                pltpu.sync_copy(rows[j], acc.at[idx_c.at[j]], add=True)
            return carry

        lax.fori_loop(0, NCH, step, 0)
        plsc.subcore_barrier()
        pltpu.sync_copy(acc.at[pl.ds(s * SPT, SPT)], out.at[pl.ds(s * SPT, SPT)])

    @pl.when(c == 0)
    def _():
        work(x_lo, out_lo)

    @pl.when(c == 1)
    def _():
        work(x_hi, out_hi)


@functools.partial(
    pl.kernel,
    out_type=(jax.ShapeDtypeStruct((SROWS, H), jnp.float32),
              jax.ShapeDtypeStruct((SROWS, H), jnp.float32)),
    mesh=_MESH,
    compiler_params=pltpu.CompilerParams(use_tc_tiling_on_sc=False),
    scratch_types=[
        pltpu.VMEM((CHK, G), jnp.int32),
        pltpu.VMEM((CHK, G), jnp.int32),
        pltpu.VMEM((G, H), jnp.float32),
        pltpu.VMEM((G, H), jnp.float32),
        pltpu.VMEM((G, H), jnp.float32),
        pltpu.VMEM((G, H), jnp.float32),
        pltpu.SemaphoreType.DMA,
        pltpu.SemaphoreType.DMA,
        pltpu.VMEM_SHARED((SROWS, H), jnp.float32),
    ],
)
def _sc_layer(x_lo, x_hi, src_g, cid_g, zeros, out_lo, out_hi,
              idx_s, idx_c, r0, r1, r2, r3, sem_i, sem_g, acc):
    _sc_layer_body(x_lo, x_hi, src_g, cid_g, zeros, out_lo, out_hi,
                   idx_s, idx_c, r0, r1, r2, r3, sem_i, sem_g, acc)


@functools.partial(
    pl.kernel,
    out_type=(jax.ShapeDtypeStruct((CROWS, H), jnp.float32),
              jax.ShapeDtypeStruct((CROWS, H), jnp.float32)),
    mesh=_MESH,
    compiler_params=pltpu.CompilerParams(use_tc_tiling_on_sc=False),
    scratch_types=[
        pltpu.VMEM((CG, G), jnp.int32),
        pltpu.VMEM((G, H), jnp.float32),
        pltpu.VMEM_SHARED((CROWS, H), jnp.float32),
    ],
)
def _sc_count(dst_g, zeros, ones, out_a, out_b, idx_d, ones_v, acc):
    c = lax.axis_index("c")
    s = lax.axis_index("s")
    w = s * NC + c
    pltpu.sync_copy(zeros.at[pl.ds(0, CPT)], acc.at[pl.ds(s * CPT, CPT)])
    pltpu.sync_copy(dst_g.at[w], idx_d)
    pltpu.sync_copy(ones, ones_v)
    plsc.subcore_barrier()

    def step(j, carry):
        pltpu.sync_copy(ones_v, acc.at[idx_d.at[j]], add=True)
        return carry

    lax.fori_loop(0, CG, step, 0)
    plsc.subcore_barrier()

    @pl.when(c == 0)
    def _():
        pltpu.sync_copy(acc.at[pl.ds(s * CPT, CPT)], out_a.at[pl.ds(s * CPT, CPT)])

    @pl.when(c == 1)
    def _():
        pltpu.sync_copy(acc.at[pl.ds(s * CPT, CPT)], out_b.at[pl.ds(s * CPT, CPT)])


@functools.partial(
    pl.kernel,
    out_type=tuple(jax.ShapeDtypeStruct((AP, H), jnp.float32) for _ in range(8)),
    mesh=_MESH,
    compiler_params=pltpu.CompilerParams(use_tc_tiling_on_sc=False),
    scratch_types=[
        pltpu.VMEM((AG, G), jnp.int32),
        pltpu.VMEM((G, H), jnp.float32),
        pltpu.SemaphoreType.DMA,
    ],
)
def _sc_readout(t0, t1, t2, t3, t4, t5, t6, t7, av_g,
                o0, o1, o2, o3, o4, o5, o6, o7, idx_v, rows, sem):
    c = lax.axis_index("c")
    s = lax.axis_index("s")
    w = s * NC + c
    pltpu.sync_copy(av_g.at[w], idx_v)
    tables = (t0, t1, t2, t3, t4, t5, t6, t7)
    outs = (o0, o1, o2, o3, o4, o5, o6, o7)

    def step(j, carry):
        base = w * APW + j * G
        for t, o in zip(tables, outs):
            pltpu.async_copy(t.at[idx_v.at[j]], rows, sem).wait()
            pltpu.sync_copy(rows, o.at[pl.ds(base, G)])
        return carry

    lax.fori_loop(0, AG, step, 0)


BM = 1000
BF = 1024


def _tc_mlp_body(x_ref, w1_ref, b1_ref, w2_ref, b2_ref, o_ref):
    h = jnp.dot(x_ref[...], w1_ref[0], preferred_element_type=jnp.float32)
    h = jnp.maximum(h + b1_ref[0], 0.0)
    o_ref[...] = jnp.dot(h, w2_ref[0], preferred_element_type=jnp.float32) + b2_ref[0]


def _tc_mlp(feats, w1s, b1s, w2s, b2s):
    grid = feats.shape[0] // BM
    half = grid // 2
    return pl.pallas_call(
        _tc_mlp_body,
        grid=(grid,),
        in_specs=[
            pl.BlockSpec((BM, 8), lambda i: (i, 0)),
            pl.BlockSpec((1, 8, D), lambda i: (i // half, 0, 0)),
            pl.BlockSpec((1, 1, D), lambda i: (i // half, 0, 0)),
            pl.BlockSpec((1, D, D), lambda i: (i // half, 0, 0)),
            pl.BlockSpec((1, 1, D), lambda i: (i // half, 0, 0)),
        ],
        out_specs=pl.BlockSpec((BM, D), lambda i: (i, 0)),
        out_shape=jax.ShapeDtypeStruct((feats.shape[0], D), jnp.float32),
    )(feats, w1s, b1s, w2s, b2s)


def _tc_combine_body(slo0_ref, shi0_ref, slo1_ref, shi1_ref, xlo_ref, xhi_ref,
                     ca_ref, cb_ref, basis_ref, att_ref, root_ref, bias_ref,
                     ylo_ref, yhi_ref):
    a0 = att_ref[0, 0]
    a1 = att_ref[1, 0]
    basis = basis_ref[...]
    s0 = jnp.concatenate([slo0_ref[...], shi0_ref[...]], axis=1)
    s1 = jnp.concatenate([slo1_ref[...], shi1_ref[...]], axis=1)
    acc = jnp.dot(s0, a0 * basis, preferred_element_type=jnp.float32)
    acc += jnp.dot(s1, a1 * basis, preferred_element_type=jnp.float32)
    cnt = ca_ref[:, :1] + cb_ref[:, :1]
    acc = acc / jnp.maximum(cnt, 1.0)
    x = jnp.concatenate([xlo_ref[...], xhi_ref[...]], axis=1)
    y = acc + jnp.dot(x, root_ref[...], preferred_element_type=jnp.float32)
    y = jnp.maximum(y + bias_ref[0], 0.0)
    ylo_ref[...] = y[:, :H]
    yhi_ref[...] = y[:, H:]


def _tc_combine(s_lo, s_hi, x_lo, x_hi, cnt_a, cnt_b, basis, att, root, bias):
    grid = N // BM
    off = N // BM
    return pl.pallas_call(
        _tc_combine_body,
        grid=(grid,),
        in_specs=[
            pl.BlockSpec((BM, H), lambda i: (i, 0)),
            pl.BlockSpec((BM, H), lambda i: (i, 0)),
            pl.BlockSpec((BM, H), lambda i: (i + off, 0)),
            pl.BlockSpec((BM, H), lambda i: (i + off, 0)),
            pl.BlockSpec((BM, H), lambda i: (i, 0)),
            pl.BlockSpec((BM, H), lambda i: (i, 0)),
            pl.BlockSpec((BM, H), lambda i: (i, 0)),
            pl.BlockSpec((BM, H), lambda i: (i, 0)),
            pl.BlockSpec((D, D), lambda i: (0, 0)),
            pl.BlockSpec(memory_space=pltpu.SMEM),
            pl.BlockSpec((D, D), lambda i: (0, 0)),
            pl.BlockSpec((1, D), lambda i: (0, 0)),
        ],
        out_specs=(pl.BlockSpec((BM, H), lambda i: (i, 0)),
                   pl.BlockSpec((BM, H), lambda i: (i, 0))),
        out_shape=(jax.ShapeDtypeStruct((N, H), jnp.float32),
                   jax.ShapeDtypeStruct((N, H), jnp.float32)),
    )(s_lo, s_hi, s_lo, s_hi, x_lo, x_hi, cnt_a, cnt_b, basis, att, root, bias)


def _tc_final_body(g0, g1, g2, g3, g4, g5, g6, g7, w1_ref, b1_ref, w4_ref,
                   b4_ref, o_ref):
    g = jnp.concatenate([g0[...], g1[...], g2[...], g3[...],
                         g4[...], g5[...], g6[...], g7[...]], axis=1)
    h = jnp.dot(g, w1_ref[...], preferred_element_type=jnp.float32)
    h = jnp.maximum(h + b1_ref[0], 0.0)
    o_ref[...] = jnp.dot(h, w4_ref[...], preferred_element_type=jnp.float32) \
        + b4_ref[0, 0]


def _tc_final(gs, fc1_W, fc1_b, fc4_W, fc4_b):
    grid = AP // BF
    return pl.pallas_call(
        _tc_final_body,
        grid=(grid,),
        in_specs=[pl.BlockSpec((BF, H), lambda i: (i, 0)) for _ in range(8)]
        + [
            pl.BlockSpec((4 * D, D), lambda i: (0, 0)),
            pl.BlockSpec((1, D), lambda i: (0, 0)),
            pl.BlockSpec((D, 1), lambda i: (0, 0)),
            pl.BlockSpec(memory_space=pltpu.SMEM),
        ],
        out_specs=pl.BlockSpec((BF, 1), lambda i: (i, 0)),
        out_shape=jax.ShapeDtypeStruct((AP, 1), jnp.float32),
    )(*gs, fc1_W, fc1_b.reshape(1, D), fc4_W, fc4_b.reshape(1, 1))


def kernel(var_node_features, con_node_features, node_types, assoc_var,
           assoc_con, edge_index, edge_types, edge_features, var_W1, var_b1,
           var_W2, var_b2, con_W1, con_b1, con_W2, con_b2, c1_basis, c1_att,
           c1_root, c1_bias, c2_basis, c2_att, c2_root, c2_bias, c3_basis,
           c3_att, c3_root, c3_bias, fc1_W, fc1_b, fc4_W, fc4_b):
    i32 = jnp.int32
    src = edge_index[0].astype(i32)
    dst = edge_index[1].astype(i32)
    et = edge_types.astype(i32)

    pad = EP - E
    srcp = jnp.concatenate([src, jnp.zeros((pad,), i32)]).reshape(NT * LG, G)
    cidx = jnp.concatenate([dst + N * et, jnp.full((pad,), TRASH, i32)])
    cidxp = cidx.reshape(NT * LG, G)
    dstp = jnp.concatenate([dst, jnp.full((pad,), CTRASH, i32)])
    dstp = dstp.reshape(NT * NC, CG, G)
    avp = jnp.concatenate(
        [assoc_var.astype(i32), jnp.zeros((AP - N // 2,), i32)]
    ).reshape(NT * NC, AG, G)

    zeros = jnp.zeros((SPT, H), jnp.float32)
    ones = jnp.ones((G, H), jnp.float32)

    feats = jnp.concatenate([var_node_features, con_node_features], axis=0)
    feats = jnp.pad(feats, ((0, 0), (0, 6)))
    w1s = jnp.stack([jnp.pad(var_W1, ((0, 6), (0, 0))),
                     jnp.pad(con_W1, ((0, 6), (0, 0)))])
    b1s = jnp.stack([var_b1.reshape(1, D), con_b1.reshape(1, D)])
    w2s = jnp.stack([var_W2, con_W2])
    b2s = jnp.stack([var_b2.reshape(1, D), con_b2.reshape(1, D)])
    y = _tc_mlp(feats, w1s, b1s, w2s, b2s)

    x0 = jnp.zeros((N, D), jnp.float32)
    x0 = x0.at[assoc_var].set(y[: N // 2])
    x0 = x0.at[assoc_con].set(y[N // 2:])
    x_lo, x_hi = x0[:, :H], x0[:, H:]

    cnt_a, cnt_b = _sc_count(dstp, zeros, ones)

    halves = [(x_lo, x_hi)]
    for basis, att, root, bias in (
        (c1_basis, c1_att, c1_root, c1_bias),
        (c2_basis, c2_att, c2_root, c2_bias),
        (c3_basis, c3_att, c3_root, c3_bias),
    ):
        lo, hi = halves[-1]
        s_lo, s_hi = _sc_layer(lo, hi, srcp, cidxp, zeros)
        halves.append(_tc_combine(
            s_lo, s_hi, lo, hi, cnt_a, cnt_b,
            basis.reshape(D, D), att.reshape(2, 1), root, bias.reshape(1, D)))

    tables = [t for pair in halves for t in pair]
    gs = _sc_readout(*tables, avp)
    out = _tc_final(gs, fc1_W, fc1_b, fc4_W, fc4_b)
    return out[: N // 2, 0]


# speedup vs baseline: 9.2902x; 1.1788x over previous
import functools

import jax
import jax.numpy as jnp
from jax import lax
from jax.experimental import pallas as pl
from jax.experimental.pallas import tpu as pltpu
from jax.experimental.pallas import tpu_sc as plsc

N = 50000
E = 800000
D = 32
H = 16
NT = 16
NC = 2
G = 128

EP = 802816
LG = EP // NT // G
CG = EP // (NT * NC) // G

SROWS = 100352
SPT = SROWS // NT
TRASH = 2 * N

CROWS = 50048
CPT = CROWS // NT
CTRASH = N

AP = 28672
AG = AP // (NT * NC) // G
APW = AP // (NT * NC)

_MESH = plsc.VectorSubcoreMesh(core_axis_name="c", subcore_axis_name="s")


CHK = 4
NCH = LG // CHK


def _sc_layer_body(x_lo, x_hi, src_g, cid_g, zeros, out_lo, out_hi,
                   ia, ca, ib, cb, ra, rb, sem_ia, sem_ib, sem_ga, sem_gb,
                   acc):
    c = lax.axis_index("c")
    s = lax.axis_index("s")
    pltpu.sync_copy(zeros, acc.at[pl.ds(s * SPT, SPT)])
    plsc.subcore_barrier()

    def work(table, out):
        def idx_start(t, idx_s, idx_c, sem):
            g0 = s * LG + t * CHK
            pltpu.make_async_copy(src_g.at[pl.ds(g0, CHK)], idx_s, sem).start()
            pltpu.make_async_copy(cid_g.at[pl.ds(g0, CHK)], idx_c, sem).start()

        def idx_wait(idx_s, idx_c, sem):
            pltpu.make_async_copy(src_g.at[pl.ds(0, CHK)], idx_s, sem).wait()
            pltpu.make_async_copy(cid_g.at[pl.ds(0, CHK)], idx_c, sem).wait()

        def gathers_start(idx_s, rows, sem):
            for j in range(CHK):
                pltpu.make_async_copy(
                    table.at[idx_s.at[j]], rows.at[j], sem).start()

        def drain(idx_c, rows, sem):
            for j in range(CHK):
                pltpu.make_async_copy(
                    table.at[idx_c.at[j]], rows.at[j], sem).wait()
            for j in range(CHK):
                pltpu.sync_copy(rows.at[j], acc.at[idx_c.at[j]], add=True)

        idx_start(0, ia, ca, sem_ia)
        idx_wait(ia, ca, sem_ia)
        gathers_start(ia, ra, sem_ga)
        idx_start(1, ib, cb, sem_ib)

        def step(i, carry):
            idx_wait(ib, cb, sem_ib)
            gathers_start(ib, rb, sem_gb)

            @pl.when(i + 1 < NCH // 2)
            def _():
                idx_start(2 * i + 2, ia, ca, sem_ia)

            drain(ca, ra, sem_ga)

            @pl.when(i + 1 < NCH // 2)
            def _():
                idx_wait(ia, ca, sem_ia)
                gathers_start(ia, ra, sem_ga)

            drain(cb, rb, sem_gb)

            @pl.when(i + 1 < NCH // 2)
            def _():
                idx_start(2 * i + 3, ib, cb, sem_ib)

            return carry

        lax.fori_loop(0, NCH // 2, step, 0)
        plsc.subcore_barrier()
        pltpu.sync_copy(acc.at[pl.ds(s * SPT, SPT)], out.at[pl.ds(s * SPT, SPT)])

    @pl.when(c == 0)
    def _():
        work(x_lo, out_lo)

    @pl.when(c == 1)
    def _():
        work(x_hi, out_hi)


@functools.partial(
    pl.kernel,
    out_type=(jax.ShapeDtypeStruct((SROWS, H), jnp.float32),
              jax.ShapeDtypeStruct((SROWS, H), jnp.float32)),
    mesh=_MESH,
    compiler_params=pltpu.CompilerParams(use_tc_tiling_on_sc=False),
    scratch_types=[
        pltpu.VMEM((CHK, G), jnp.int32),
        pltpu.VMEM((CHK, G), jnp.int32),
        pltpu.VMEM((CHK, G), jnp.int32),
        pltpu.VMEM((CHK, G), jnp.int32),
        pltpu.VMEM((CHK, G, H), jnp.float32),
        pltpu.VMEM((CHK, G, H), jnp.float32),
        pltpu.SemaphoreType.DMA,
        pltpu.SemaphoreType.DMA,
        pltpu.SemaphoreType.DMA,
        pltpu.SemaphoreType.DMA,
        pltpu.VMEM_SHARED((SROWS, H), jnp.float32),
    ],
)
def _sc_layer(x_lo, x_hi, src_g, cid_g, zeros, out_lo, out_hi,
              ia, ca, ib, cb, ra, rb, sem_ia, sem_ib, sem_ga, sem_gb, acc):
    _sc_layer_body(x_lo, x_hi, src_g, cid_g, zeros, out_lo, out_hi,
                   ia, ca, ib, cb, ra, rb, sem_ia, sem_ib, sem_ga, sem_gb,
                   acc)


@functools.partial(
    pl.kernel,
    out_type=(jax.ShapeDtypeStruct((CROWS, H), jnp.float32),
              jax.ShapeDtypeStruct((CROWS, H), jnp.float32)),
    mesh=_MESH,
    compiler_params=pltpu.CompilerParams(use_tc_tiling_on_sc=False),
    scratch_types=[
        pltpu.VMEM((CG, G), jnp.int32),
        pltpu.VMEM((G, H), jnp.float32),
        pltpu.VMEM_SHARED((CROWS, H), jnp.float32),
    ],
)
def _sc_count(dst_g, zeros, ones, out_a, out_b, idx_d, ones_v, acc):
    c = lax.axis_index("c")
    s = lax.axis_index("s")
    w = s * NC + c
    pltpu.sync_copy(zeros.at[pl.ds(0, CPT)], acc.at[pl.ds(s * CPT, CPT)])
    pltpu.sync_copy(dst_g.at[w], idx_d)
    pltpu.sync_copy(ones, ones_v)
    plsc.subcore_barrier()

    def step(j, carry):
        pltpu.sync_copy(ones_v, acc.at[idx_d.at[j]], add=True)
        return carry

    lax.fori_loop(0, CG, step, 0)
    plsc.subcore_barrier()

    @pl.when(c == 0)
    def _():
        pltpu.sync_copy(acc.at[pl.ds(s * CPT, CPT)], out_a.at[pl.ds(s * CPT, CPT)])

    @pl.when(c == 1)
    def _():
        pltpu.sync_copy(acc.at[pl.ds(s * CPT, CPT)], out_b.at[pl.ds(s * CPT, CPT)])


@functools.partial(
    pl.kernel,
    out_type=tuple(jax.ShapeDtypeStruct((AP, H), jnp.float32) for _ in range(8)),
    mesh=_MESH,
    compiler_params=pltpu.CompilerParams(use_tc_tiling_on_sc=False),
    scratch_types=[
        pltpu.VMEM((AG, G), jnp.int32),
        pltpu.VMEM((G, H), jnp.float32),
        pltpu.SemaphoreType.DMA,
    ],
)
def _sc_readout(t0, t1, t2, t3, t4, t5, t6, t7, av_g,
                o0, o1, o2, o3, o4, o5, o6, o7, idx_v, rows, sem):
    c = lax.axis_index("c")
    s = lax.axis_index("s")
    w = s * NC + c
    pltpu.sync_copy(av_g.at[w], idx_v)
    tables = (t0, t1, t2, t3, t4, t5, t6, t7)
    outs = (o0, o1, o2, o3, o4, o5, o6, o7)

    def step(j, carry):
        base = w * APW + j * G
        for t, o in zip(tables, outs):
            pltpu.async_copy(t.at[idx_v.at[j]], rows, sem).wait()
            pltpu.sync_copy(rows, o.at[pl.ds(base, G)])
        return carry

    lax.fori_loop(0, AG, step, 0)


BM = 1000
BC = 2000
BF = 1024


def _tc_mlp_body(x_ref, w1_ref, b1_ref, w2_ref, b2_ref, o_ref):
    h = jnp.dot(x_ref[...], w1_ref[0], preferred_element_type=jnp.float32)
    h = jnp.maximum(h + b1_ref[0], 0.0)
    o_ref[...] = jnp.dot(h, w2_ref[0], preferred_element_type=jnp.float32) + b2_ref[0]


def _tc_mlp(feats, w1s, b1s, w2s, b2s):
    grid = feats.shape[0] // BM
    half = grid // 2
    return pl.pallas_call(
        _tc_mlp_body,
        grid=(grid,),
        in_specs=[
            pl.BlockSpec((BM, 8), lambda i: (i, 0)),
            pl.BlockSpec((1, 8, D), lambda i: (i // half, 0, 0)),
            pl.BlockSpec((1, 1, D), lambda i: (i // half, 0, 0)),
            pl.BlockSpec((1, D, D), lambda i: (i // half, 0, 0)),
            pl.BlockSpec((1, 1, D), lambda i: (i // half, 0, 0)),
        ],
        out_specs=pl.BlockSpec((BM, D), lambda i: (i, 0)),
        out_shape=jax.ShapeDtypeStruct((feats.shape[0], D), jnp.float32),
    )(feats, w1s, b1s, w2s, b2s)


def _tc_combine_body(slo0_ref, shi0_ref, slo1_ref, shi1_ref, xlo_ref, xhi_ref,
                     ca_ref, cb_ref, basis_ref, att_ref, root_ref, bias_ref,
                     ylo_ref, yhi_ref):
    a0 = att_ref[0, 0]
    a1 = att_ref[1, 0]
    basis = basis_ref[...]
    s0 = jnp.concatenate([slo0_ref[...], shi0_ref[...]], axis=1)
    s1 = jnp.concatenate([slo1_ref[...], shi1_ref[...]], axis=1)
    acc = jnp.dot(s0, a0 * basis, preferred_element_type=jnp.float32)
    acc += jnp.dot(s1, a1 * basis, preferred_element_type=jnp.float32)
    cnt = ca_ref[:, :1] + cb_ref[:, :1]
    acc = acc / jnp.maximum(cnt, 1.0)
    x = jnp.concatenate([xlo_ref[...], xhi_ref[...]], axis=1)
    y = acc + jnp.dot(x, root_ref[...], preferred_element_type=jnp.float32)
    y = jnp.maximum(y + bias_ref[0], 0.0)
    ylo_ref[...] = y[:, :H]
    yhi_ref[...] = y[:, H:]


def _tc_combine(s_lo, s_hi, x_lo, x_hi, cnt_a, cnt_b, basis, att, root, bias):
    grid = N // BC
    off = N // BC
    return pl.pallas_call(
        _tc_combine_body,
        grid=(grid,),
        in_specs=[
            pl.BlockSpec((BC, H), lambda i: (i, 0)),
            pl.BlockSpec((BC, H), lambda i: (i, 0)),
            pl.BlockSpec((BC, H), lambda i: (i + off, 0)),
            pl.BlockSpec((BC, H), lambda i: (i + off, 0)),
            pl.BlockSpec((BC, H), lambda i: (i, 0)),
            pl.BlockSpec((BC, H), lambda i: (i, 0)),
            pl.BlockSpec((BC, H), lambda i: (i, 0)),
            pl.BlockSpec((BC, H), lambda i: (i, 0)),
            pl.BlockSpec((D, D), lambda i: (0, 0)),
            pl.BlockSpec(memory_space=pltpu.SMEM),
            pl.BlockSpec((D, D), lambda i: (0, 0)),
            pl.BlockSpec((1, D), lambda i: (0, 0)),
        ],
        out_specs=(pl.BlockSpec((BC, H), lambda i: (i, 0)),
                   pl.BlockSpec((BC, H), lambda i: (i, 0))),
        out_shape=(jax.ShapeDtypeStruct((N, H), jnp.float32),
                   jax.ShapeDtypeStruct((N, H), jnp.float32)),
    )(s_lo, s_hi, s_lo, s_hi, x_lo, x_hi, cnt_a, cnt_b, basis, att, root, bias)


def _tc_final_body(g0, g1, g2, g3, g4, g5, g6, g7, w1_ref, b1_ref, w4_ref,
                   b4_ref, o_ref):
    g = jnp.concatenate([g0[...], g1[...], g2[...], g3[...],
                         g4[...], g5[...], g6[...], g7[...]], axis=1)
    h = jnp.dot(g, w1_ref[...], preferred_element_type=jnp.float32)
    h = jnp.maximum(h + b1_ref[0], 0.0)
    o_ref[...] = jnp.dot(h, w4_ref[...], preferred_element_type=jnp.float32) \
        + b4_ref[0, 0]


def _tc_final(gs, fc1_W, fc1_b, fc4_W, fc4_b):
    grid = AP // BF
    return pl.pallas_call(
        _tc_final_body,
        grid=(grid,),
        in_specs=[pl.BlockSpec((BF, H), lambda i: (i, 0)) for _ in range(8)]
        + [
            pl.BlockSpec((4 * D, D), lambda i: (0, 0)),
            pl.BlockSpec((1, D), lambda i: (0, 0)),
            pl.BlockSpec((D, 1), lambda i: (0, 0)),
            pl.BlockSpec(memory_space=pltpu.SMEM),
        ],
        out_specs=pl.BlockSpec((BF, 1), lambda i: (i, 0)),
        out_shape=jax.ShapeDtypeStruct((AP, 1), jnp.float32),
    )(*gs, fc1_W, fc1_b.reshape(1, D), fc4_W, fc4_b.reshape(1, 1))


def kernel(var_node_features, con_node_features, node_types, assoc_var,
           assoc_con, edge_index, edge_types, edge_features, var_W1, var_b1,
           var_W2, var_b2, con_W1, con_b1, con_W2, con_b2, c1_basis, c1_att,
           c1_root, c1_bias, c2_basis, c2_att, c2_root, c2_bias, c3_basis,
           c3_att, c3_root, c3_bias, fc1_W, fc1_b, fc4_W, fc4_b):
    i32 = jnp.int32
    src = edge_index[0].astype(i32)
    dst = edge_index[1].astype(i32)
    et = edge_types.astype(i32)

    pad = EP - E
    srcp = jnp.concatenate([src, jnp.zeros((pad,), i32)]).reshape(NT * LG, G)
    cidx = jnp.concatenate([dst + N * et, jnp.full((pad,), TRASH, i32)])
    cidxp = cidx.reshape(NT * LG, G)
    dstp = jnp.concatenate([dst, jnp.full((pad,), CTRASH, i32)])
    dstp = dstp.reshape(NT * NC, CG, G)
    avp = jnp.concatenate(
        [assoc_var.astype(i32), jnp.zeros((AP - N // 2,), i32)]
    ).reshape(NT * NC, AG, G)

    zeros = jnp.zeros((SPT, H), jnp.float32)
    ones = jnp.ones((G, H), jnp.float32)

    feats = jnp.concatenate([var_node_features, con_node_features], axis=0)
    feats = jnp.pad(feats, ((0, 0), (0, 6)))
    w1s = jnp.stack([jnp.pad(var_W1, ((0, 6), (0, 0))),
                     jnp.pad(con_W1, ((0, 6), (0, 0)))])
    b1s = jnp.stack([var_b1.reshape(1, D), con_b1.reshape(1, D)])
    w2s = jnp.stack([var_W2, con_W2])
    b2s = jnp.stack([var_b2.reshape(1, D), con_b2.reshape(1, D)])
    y = _tc_mlp(feats, w1s, b1s, w2s, b2s)

    x0 = jnp.zeros((N, D), jnp.float32)
    x0 = x0.at[jnp.concatenate([assoc_var, assoc_con])].set(y)
    x_lo, x_hi = x0[:, :H], x0[:, H:]

    cnt_a, cnt_b = _sc_count(dstp, zeros, ones)

    halves = [(x_lo, x_hi)]
    for basis, att, root, bias in (
        (c1_basis, c1_att, c1_root, c1_bias),
        (c2_basis, c2_att, c2_root, c2_bias),
        (c3_basis, c3_att, c3_root, c3_bias),
    ):
        lo, hi = halves[-1]
        s_lo, s_hi = _sc_layer(lo, hi, srcp, cidxp, zeros)
        halves.append(_tc_combine(
            s_lo, s_hi, lo, hi, cnt_a, cnt_b,
            basis.reshape(D, D), att.reshape(2, 1), root, bias.reshape(1, D)))

    tables = [t for pair in halves for t in pair]
    gs = _sc_readout(*tables, avp)
    out = _tc_final(gs, fc1_W, fc1_b, fc4_W, fc4_b)
    return out[: N // 2, 0]


# speedup vs baseline: 13.2636x; 1.4277x over previous
import functools

import jax
import jax.numpy as jnp
from jax import lax
from jax.experimental import pallas as pl
from jax.experimental.pallas import tpu as pltpu
from jax.experimental.pallas import tpu_sc as plsc

N = 50000
N2 = 51200
E = 800000
D = 32
H = 16
NT = 16
NC = 2
G = 128

EP = 802816
LG = EP // NT // G
CG = EP // (NT * NC) // G

SROWS = 2 * N2
SPT = SROWS // NT
TRASH = 50048

CROWS = N2
CPT = CROWS // NT

AP = 28672
AG = AP // (NT * NC) // G
APW = AP // (NT * NC)

XP = N2 // 4
SP4 = SROWS // 4
AP4 = AP // 4

_MESH = plsc.VectorSubcoreMesh(core_axis_name="c", subcore_axis_name="s")


CHK = 4
NCH = LG // CHK


def _sc_layer_body(xv, src_lo, src_hi, cid_g, zeros, out,
                   ia, ca, ib, cb, ra, rb, sem_ia, sem_ib, sem_ga, sem_gb,
                   acc):
    c = lax.axis_index("c")
    s = lax.axis_index("s")
    pltpu.sync_copy(zeros, acc.at[pl.ds(s * SPT, SPT)])
    plsc.subcore_barrier()

    def work(src_g, col):
        def idx_start(t, idx_s, idx_c, sem):
            g0 = s * LG + t * CHK
            pltpu.make_async_copy(src_g.at[pl.ds(g0, CHK)], idx_s, sem).start()
            pltpu.make_async_copy(cid_g.at[pl.ds(g0, CHK)], idx_c, sem).start()

        def idx_wait(idx_s, idx_c, sem):
            pltpu.make_async_copy(src_g.at[pl.ds(0, CHK)], idx_s, sem).wait()
            pltpu.make_async_copy(cid_g.at[pl.ds(0, CHK)], idx_c, sem).wait()

        def gathers_start(idx_s, rows, sem):
            for j in range(CHK):
                pltpu.make_async_copy(
                    xv.at[idx_s.at[j]], rows.at[j], sem).start()

        def drain(idx_c, rows, sem):
            for j in range(CHK):
                pltpu.make_async_copy(
                    xv.at[idx_c.at[j]], rows.at[j], sem).wait()
            for j in range(CHK):
                pltpu.sync_copy(rows.at[j], acc.at[idx_c.at[j]], add=True)

        idx_start(0, ia, ca, sem_ia)
        idx_wait(ia, ca, sem_ia)
        gathers_start(ia, ra, sem_ga)
        idx_start(1, ib, cb, sem_ib)

        def step(i, carry):
            idx_wait(ib, cb, sem_ib)
            gathers_start(ib, rb, sem_gb)

            @pl.when(i + 1 < NCH // 2)
            def _():
                idx_start(2 * i + 2, ia, ca, sem_ia)

            drain(ca, ra, sem_ga)

            @pl.when(i + 1 < NCH // 2)
            def _():
                idx_wait(ia, ca, sem_ia)
                gathers_start(ia, ra, sem_ga)

            drain(cb, rb, sem_gb)

            @pl.when(i + 1 < NCH // 2)
            def _():
                idx_start(2 * i + 3, ib, cb, sem_ib)

            return carry

        lax.fori_loop(0, NCH // 2, step, 0)
        plsc.subcore_barrier()
        pltpu.sync_copy(acc.at[pl.ds(s * SPT, SPT)],
                        out.at[pl.ds(s * SPT, SPT), pl.ds(col, H)])

    @pl.when(c == 0)
    def _():
        work(src_lo, 0)

    @pl.when(c == 1)
    def _():
        work(src_hi, H)


@functools.partial(
    pl.kernel,
    out_type=jax.ShapeDtypeStruct((SROWS, D), jnp.float32),
    mesh=_MESH,
    compiler_params=pltpu.CompilerParams(use_tc_tiling_on_sc=False),
    scratch_types=[
        pltpu.VMEM((CHK, G), jnp.int32),
        pltpu.VMEM((CHK, G), jnp.int32),
        pltpu.VMEM((CHK, G), jnp.int32),
        pltpu.VMEM((CHK, G), jnp.int32),
        pltpu.VMEM((CHK, G, H), jnp.float32),
        pltpu.VMEM((CHK, G, H), jnp.float32),
        pltpu.SemaphoreType.DMA,
        pltpu.SemaphoreType.DMA,
        pltpu.SemaphoreType.DMA,
        pltpu.SemaphoreType.DMA,
        pltpu.VMEM_SHARED((SROWS, H), jnp.float32),
    ],
)
def _sc_layer(xv, src_lo, src_hi, cid_g, zeros, out,
              ia, ca, ib, cb, ra, rb, sem_ia, sem_ib, sem_ga, sem_gb, acc):
    _sc_layer_body(xv, src_lo, src_hi, cid_g, zeros, out,
                   ia, ca, ib, cb, ra, rb, sem_ia, sem_ib, sem_ga, sem_gb,
                   acc)


@functools.partial(
    pl.kernel,
    out_type=jax.ShapeDtypeStruct((CROWS, D), jnp.float32),
    mesh=_MESH,
    compiler_params=pltpu.CompilerParams(use_tc_tiling_on_sc=False),
    scratch_types=[
        pltpu.VMEM((CG, G), jnp.int32),
        pltpu.VMEM((G, H), jnp.float32),
        pltpu.VMEM_SHARED((CROWS, H), jnp.float32),
    ],
)
def _sc_count(dst_g, zeros, ones, out, idx_d, ones_v, acc):
    c = lax.axis_index("c")
    s = lax.axis_index("s")
    w = s * NC + c
    pltpu.sync_copy(zeros.at[pl.ds(0, CPT)], acc.at[pl.ds(s * CPT, CPT)])
    pltpu.sync_copy(dst_g.at[w], idx_d)
    pltpu.sync_copy(ones, ones_v)
    plsc.subcore_barrier()

    def step(j, carry):
        pltpu.sync_copy(ones_v, acc.at[idx_d.at[j]], add=True)
        return carry

    lax.fori_loop(0, CG, step, 0)
    plsc.subcore_barrier()

    @pl.when(c == 0)
    def _():
        pltpu.sync_copy(acc.at[pl.ds(s * CPT, CPT)],
                        out.at[pl.ds(s * CPT, CPT), pl.ds(0, H)])

    @pl.when(c == 1)
    def _():
        pltpu.sync_copy(acc.at[pl.ds(s * CPT, CPT)],
                        out.at[pl.ds(s * CPT, CPT), pl.ds(H, H)])


@functools.partial(
    pl.kernel,
    out_type=tuple(jax.ShapeDtypeStruct((AP, D), jnp.float32) for _ in range(4)),
    mesh=_MESH,
    compiler_params=pltpu.CompilerParams(use_tc_tiling_on_sc=False),
    scratch_types=[
        pltpu.VMEM((AG, G), jnp.int32),
        pltpu.VMEM((4, G, D), jnp.float32),
        pltpu.VMEM((4, G, D), jnp.float32),
        pltpu.SemaphoreType.DMA,
        pltpu.SemaphoreType.DMA,
    ],
)
def _sc_readout(t0, t1, t2, t3, av_g, o0, o1, o2, o3,
                idx_v, ra, rb, sem_a, sem_b):
    c = lax.axis_index("c")
    s = lax.axis_index("s")
    w = s * NC + c
    pltpu.sync_copy(av_g.at[w], idx_v)
    tables = (t0, t1, t2, t3)
    outs = (o0, o1, o2, o3)

    def start_all(j, rows, sem):
        for k, t in enumerate(tables):
            pltpu.make_async_copy(t.at[idx_v.at[j]], rows.at[k], sem).start()

    def drain_all(j, rows, sem):
        for k, t in enumerate(tables):
            pltpu.make_async_copy(t.at[idx_v.at[j]], rows.at[k], sem).wait()
        for k, o in enumerate(outs):
            pltpu.sync_copy(rows.at[k], o.at[pl.ds(w * APW + j * G, G)])

    start_all(0, ra, sem_a)

    def step(i, carry):
        start_all(2 * i + 1, rb, sem_b)
        drain_all(2 * i, ra, sem_a)
        start_all(2 * i + 2, ra, sem_a)
        drain_all(2 * i + 1, rb, sem_b)
        return carry

    lax.fori_loop(0, (AG - 1) // 2, step, 0)
    drain_all(AG - 1, ra, sem_a)


BM = 1000
BP = 512


def _bd4(w):
    z = jnp.zeros((D, D), jnp.float32)
    rows = [jnp.concatenate([w if j == k else z for j in range(4)], axis=1)
            for k in range(4)]
    return jnp.concatenate(rows, axis=0)


def _tc_mlp_body(x_ref, w1_ref, b1_ref, w2_ref, b2_ref, o_ref):
    h = jnp.dot(x_ref[...], w1_ref[0], preferred_element_type=jnp.float32)
    h = jnp.maximum(h + b1_ref[0], 0.0)
    o_ref[...] = jnp.dot(h, w2_ref[0], preferred_element_type=jnp.float32) + b2_ref[0]


def _tc_mlp(feats, w1s, b1s, w2s, b2s):
    grid = feats.shape[0] // BM
    half = grid // 2
    return pl.pallas_call(
        _tc_mlp_body,
        grid=(grid,),
        in_specs=[
            pl.BlockSpec((BM, 8), lambda i: (i, 0)),
            pl.BlockSpec((1, 8, D), lambda i: (i // half, 0, 0)),
            pl.BlockSpec((1, 1, D), lambda i: (i // half, 0, 0)),
            pl.BlockSpec((1, D, D), lambda i: (i // half, 0, 0)),
            pl.BlockSpec((1, 1, D), lambda i: (i // half, 0, 0)),
        ],
        out_specs=pl.BlockSpec((BM, D), lambda i: (i, 0)),
        out_shape=jax.ShapeDtypeStruct((feats.shape[0], D), jnp.float32),
    )(feats, w1s, b1s, w2s, b2s)


def _tc_combine_body(s0_ref, s1_ref, xp_ref, cp_ref, basis_ref, att_ref,
                     root_ref, bias_ref, o_ref):
    a0 = att_ref[0, 0]
    a1 = att_ref[1, 0]
    basis = basis_ref[...]
    aggr = jnp.dot(s0_ref[...], _bd4(a0 * basis),
                   preferred_element_type=jnp.float32)
    aggr += jnp.dot(s1_ref[...], _bd4(a1 * basis),
                    preferred_element_type=jnp.float32)
    tot = jnp.dot(cp_ref[...], _bd4(jnp.full((D, D), 1.0 / H, jnp.float32)),
                  preferred_element_type=jnp.float32)
    aggr = aggr / jnp.maximum(tot, 1.0)
    y = aggr + jnp.dot(xp_ref[...], _bd4(root_ref[...]),
                       preferred_element_type=jnp.float32)
    o_ref[...] = jnp.maximum(y + bias_ref[0], 0.0)


def _tc_combine(sp, xp, cp, basis, att, root, bias4):
    grid = XP // BP
    off = XP // BP
    return pl.pallas_call(
        _tc_combine_body,
        grid=(grid,),
        in_specs=[
            pl.BlockSpec((BP, 128), lambda i: (i, 0)),
            pl.BlockSpec((BP, 128), lambda i: (i + off, 0)),
            pl.BlockSpec((BP, 128), lambda i: (i, 0)),
            pl.BlockSpec((BP, 128), lambda i: (i, 0)),
            pl.BlockSpec((D, D), lambda i: (0, 0)),
            pl.BlockSpec(memory_space=pltpu.SMEM),
            pl.BlockSpec((D, D), lambda i: (0, 0)),
            pl.BlockSpec((1, 128), lambda i: (0, 0)),
        ],
        out_specs=pl.BlockSpec((BP, 128), lambda i: (i, 0)),
        out_shape=jax.ShapeDtypeStruct((XP, 128), jnp.float32),
    )(sp, sp, xp, cp, basis, att, root, bias4)


def _tc_final_body(g0, g1, g2, g3, w1_ref, b1_ref, w4_ref, b4_ref, o_ref):
    gs = (g0, g1, g2, g3)
    h = None
    for k, g in enumerate(gs):
        part = jnp.dot(g[...], _bd4(w1_ref[pl.ds(k * D, D)]),
                       preferred_element_type=jnp.float32)
        h = part if h is None else h + part
    h = jnp.maximum(h + b1_ref[0], 0.0)
    z = jnp.zeros((D, 1), jnp.float32)
    w4 = w4_ref[...]
    cols = [jnp.concatenate([w4 if j == k else z for j in range(4)], axis=0)
            for k in range(4)]
    f = jnp.concatenate(cols, axis=1)
    o_ref[...] = jnp.dot(h, f, preferred_element_type=jnp.float32) \
        + b4_ref[0, 0]


def _tc_final(gs, fc1_W, fc1_b4, fc4_W, fc4_b):
    grid = AP4 // BP
    return pl.pallas_call(
        _tc_final_body,
        grid=(grid,),
        in_specs=[pl.BlockSpec((BP, 128), lambda i: (i, 0)) for _ in range(4)]
        + [
            pl.BlockSpec((4 * D, D), lambda i: (0, 0)),
            pl.BlockSpec((1, 128), lambda i: (0, 0)),
            pl.BlockSpec((D, 1), lambda i: (0, 0)),
            pl.BlockSpec(memory_space=pltpu.SMEM),
        ],
        out_specs=pl.BlockSpec((BP, 4), lambda i: (i, 0)),
        out_shape=jax.ShapeDtypeStruct((AP4, 4), jnp.float32),
    )(*gs, fc1_W, fc1_b4, fc4_W, fc4_b.reshape(1, 1))


def kernel(var_node_features, con_node_features, node_types, assoc_var,
           assoc_con, edge_index, edge_types, edge_features, var_W1, var_b1,
           var_W2, var_b2, con_W1, con_b1, con_W2, con_b2, c1_basis, c1_att,
           c1_root, c1_bias, c2_basis, c2_att, c2_root, c2_bias, c3_basis,
           c3_att, c3_root, c3_bias, fc1_W, fc1_b, fc4_W, fc4_b):
    i32 = jnp.int32
    src = edge_index[0].astype(i32)
    dst = edge_index[1].astype(i32)
    et = edge_types.astype(i32)

    pad = EP - E
    zpad = jnp.zeros((pad,), i32)
    src_lo = jnp.concatenate([2 * src, zpad]).reshape(NT * LG, G)
    src_hi = jnp.concatenate([2 * src + 1, zpad + 1]).reshape(NT * LG, G)
    cidxp = jnp.concatenate([dst + N2 * et,
                             jnp.full((pad,), TRASH, i32)]).reshape(NT * LG, G)
    dstp = jnp.concatenate([dst, jnp.full((pad,), TRASH, i32)])
    dstp = dstp.reshape(NT * NC, CG, G)
    avp = jnp.concatenate(
        [assoc_var.astype(i32), jnp.zeros((AP - N // 2,), i32)]
    ).reshape(NT * NC, AG, G)

    zeros = jnp.zeros((SPT, H), jnp.float32)
    ones = jnp.ones((G, H), jnp.float32)

    feats = jnp.concatenate([var_node_features, con_node_features], axis=0)
    feats = jnp.pad(feats, ((0, 0), (0, 6)))
    w1s = jnp.stack([jnp.pad(var_W1, ((0, 6), (0, 0))),
                     jnp.pad(con_W1, ((0, 6), (0, 0)))])
    b1s = jnp.stack([var_b1.reshape(1, D), con_b1.reshape(1, D)])
    w2s = jnp.stack([var_W2, con_W2])
    b2s = jnp.stack([var_b2.reshape(1, D), con_b2.reshape(1, D)])
    y = _tc_mlp(feats, w1s, b1s, w2s, b2s)

    x0 = jnp.zeros((N2, D), jnp.float32)
    x0 = x0.at[jnp.concatenate([assoc_var, assoc_con])].set(y)
    xp0 = x0.reshape(XP, 128)

    cnt = _sc_count(dstp, zeros, ones)
    cp = cnt.reshape(XP, 128)

    xps = [xp0]
    for basis, att, root, bias in (
        (c1_basis, c1_att, c1_root, c1_bias),
        (c2_basis, c2_att, c2_root, c2_bias),
        (c3_basis, c3_att, c3_root, c3_bias),
    ):
        xp = xps[-1]
        s_out = _sc_layer(xp.reshape(2 * N2, H), src_lo, src_hi, cidxp, zeros)
        xps.append(_tc_combine(
            s_out.reshape(SP4, 128), xp, cp,
            basis.reshape(D, D), att.reshape(2, 1), root,
            jnp.tile(bias.reshape(1, D), (1, 4))))

    tables = [xp.reshape(N2, D) for xp in xps]
    gs = _sc_readout(*tables, avp)
    out = _tc_final([g.reshape(AP4, 128) for g in gs],
                    fc1_W, jnp.tile(fc1_b.reshape(1, D), (1, 4)),
                    fc4_W, fc4_b)
    return out.reshape(AP)[: N // 2]


# speedup vs baseline: 14.0595x; 1.0600x over previous
import functools

import jax
import jax.numpy as jnp
from jax import lax
from jax.experimental import pallas as pl
from jax.experimental.pallas import tpu as pltpu
from jax.experimental.pallas import tpu_sc as plsc

N = 50000
N2 = 51200
E = 800000
D = 32
H = 16
NT = 16
NC = 2
G = 128

EP = 802816
LG = EP // NT // G
CG = EP // (NT * NC) // G

SROWS = 2 * N2
SPT = SROWS // NT
TRASH = 50048

CROWS = N2
CPT = CROWS // NT

AP = 28672
AG = AP // (NT * NC) // G
APW = AP // (NT * NC)

XP = N2 // 4
SP4 = SROWS // 4
AP4 = AP // 4

_MESH = plsc.VectorSubcoreMesh(core_axis_name="c", subcore_axis_name="s")


CHK = 4
NCH = LG // CHK


def _sc_layer_body(xv, src_lo, src_hi, cid_g, zeros, out,
                   ia, ca, ib, cb, ra, rb, sem_ia, sem_ib, sem_ga, sem_gb,
                   sem_sa, sem_sb, acc):
    c = lax.axis_index("c")
    s = lax.axis_index("s")
    pltpu.sync_copy(zeros, acc.at[pl.ds(s * SPT, SPT)])
    plsc.subcore_barrier()

    def work(src_g, col):
        def idx_start(t, idx_s, idx_c, sem):
            g0 = s * LG + t * CHK
            pltpu.make_async_copy(src_g.at[pl.ds(g0, CHK)], idx_s, sem).start()
            pltpu.make_async_copy(cid_g.at[pl.ds(g0, CHK)], idx_c, sem).start()

        def idx_wait(idx_s, idx_c, sem):
            pltpu.make_async_copy(src_g.at[pl.ds(0, CHK)], idx_s, sem).wait()
            pltpu.make_async_copy(cid_g.at[pl.ds(0, CHK)], idx_c, sem).wait()

        def gathers_start(idx_s, rows, sem):
            for j in range(CHK):
                pltpu.make_async_copy(
                    xv.at[idx_s.at[j]], rows.at[j], sem).start()

        def gathers_wait(idx_c, rows, sem):
            for j in range(CHK):
                pltpu.make_async_copy(
                    xv.at[idx_c.at[j]], rows.at[j], sem).wait()

        def scat_start(idx_c, rows, sem):
            for j in range(CHK):
                pltpu.make_async_copy(
                    rows.at[j], acc.at[idx_c.at[j]], sem).start(add=True)

        def scat_wait(idx_c, rows, sem):
            for j in range(CHK):
                pltpu.make_async_copy(
                    rows.at[j], acc.at[idx_c.at[j]], sem).wait()

        idx_start(0, ia, ca, sem_ia)
        idx_wait(ia, ca, sem_ia)
        gathers_start(ia, ra, sem_ga)
        idx_start(1, ib, cb, sem_ib)

        def step(i, carry):
            idx_wait(ib, cb, sem_ib)

            @pl.when(i > 0)
            def _():
                scat_wait(cb, rb, sem_sb)

            gathers_start(ib, rb, sem_gb)

            @pl.when(i + 1 < NCH // 2)
            def _():
                idx_start(2 * i + 2, ia, ca, sem_ia)

            gathers_wait(ca, ra, sem_ga)
            scat_start(ca, ra, sem_sa)

            @pl.when(i + 1 < NCH // 2)
            def _():
                idx_wait(ia, ca, sem_ia)
                scat_wait(ca, ra, sem_sa)
                gathers_start(ia, ra, sem_ga)

            gathers_wait(cb, rb, sem_gb)
            scat_start(cb, rb, sem_sb)

            @pl.when(i + 1 < NCH // 2)
            def _():
                idx_start(2 * i + 3, ib, cb, sem_ib)

            return carry

        lax.fori_loop(0, NCH // 2, step, 0)
        scat_wait(ca, ra, sem_sa)
        scat_wait(cb, rb, sem_sb)
        plsc.subcore_barrier()
        pltpu.sync_copy(acc.at[pl.ds(s * SPT, SPT)],
                        out.at[pl.ds(s * SPT, SPT), pl.ds(col, H)])

    @pl.when(c == 0)
    def _():
        work(src_lo, 0)

    @pl.when(c == 1)
    def _():
        work(src_hi, H)


@functools.partial(
    pl.kernel,
    out_type=jax.ShapeDtypeStruct((SROWS, D), jnp.float32),
    mesh=_MESH,
    compiler_params=pltpu.CompilerParams(use_tc_tiling_on_sc=False),
    scratch_types=[
        pltpu.VMEM((CHK, G), jnp.int32),
        pltpu.VMEM((CHK, G), jnp.int32),
        pltpu.VMEM((CHK, G), jnp.int32),
        pltpu.VMEM((CHK, G), jnp.int32),
        pltpu.VMEM((CHK, G, H), jnp.float32),
        pltpu.VMEM((CHK, G, H), jnp.float32),
        pltpu.SemaphoreType.DMA,
        pltpu.SemaphoreType.DMA,
        pltpu.SemaphoreType.DMA,
        pltpu.SemaphoreType.DMA,
        pltpu.SemaphoreType.DMA,
        pltpu.SemaphoreType.DMA,
        pltpu.VMEM_SHARED((SROWS, H), jnp.float32),
    ],
)
def _sc_layer(xv, src_lo, src_hi, cid_g, zeros, out,
              ia, ca, ib, cb, ra, rb, sem_ia, sem_ib, sem_ga, sem_gb,
              sem_sa, sem_sb, acc):
    _sc_layer_body(xv, src_lo, src_hi, cid_g, zeros, out,
                   ia, ca, ib, cb, ra, rb, sem_ia, sem_ib, sem_ga, sem_gb,
                   sem_sa, sem_sb, acc)


@functools.partial(
    pl.kernel,
    out_type=jax.ShapeDtypeStruct((CROWS, D), jnp.float32),
    mesh=_MESH,
    compiler_params=pltpu.CompilerParams(use_tc_tiling_on_sc=False),
    scratch_types=[
        pltpu.VMEM((CG, G), jnp.int32),
        pltpu.VMEM((G, H), jnp.float32),
        pltpu.VMEM_SHARED((CROWS, H), jnp.float32),
    ],
)
def _sc_count(dst_g, zeros, ones, out, idx_d, ones_v, acc):
    c = lax.axis_index("c")
    s = lax.axis_index("s")
    w = s * NC + c
    pltpu.sync_copy(zeros.at[pl.ds(0, CPT)], acc.at[pl.ds(s * CPT, CPT)])
    pltpu.sync_copy(dst_g.at[w], idx_d)
    pltpu.sync_copy(ones, ones_v)
    plsc.subcore_barrier()

    def step(j, carry):
        pltpu.sync_copy(ones_v, acc.at[idx_d.at[j]], add=True)
        return carry

    lax.fori_loop(0, CG, step, 0)
    plsc.subcore_barrier()

    @pl.when(c == 0)
    def _():
        pltpu.sync_copy(acc.at[pl.ds(s * CPT, CPT)],
                        out.at[pl.ds(s * CPT, CPT), pl.ds(0, H)])

    @pl.when(c == 1)
    def _():
        pltpu.sync_copy(acc.at[pl.ds(s * CPT, CPT)],
                        out.at[pl.ds(s * CPT, CPT), pl.ds(H, H)])


@functools.partial(
    pl.kernel,
    out_type=tuple(jax.ShapeDtypeStruct((AP, D), jnp.float32) for _ in range(4)),
    mesh=_MESH,
    compiler_params=pltpu.CompilerParams(use_tc_tiling_on_sc=False),
    scratch_types=[
        pltpu.VMEM((AG, G), jnp.int32),
        pltpu.VMEM((4, G, D), jnp.float32),
        pltpu.VMEM((4, G, D), jnp.float32),
        pltpu.SemaphoreType.DMA,
        pltpu.SemaphoreType.DMA,
    ],
)
def _sc_readout(t0, t1, t2, t3, av_g, o0, o1, o2, o3,
                idx_v, ra, rb, sem_a, sem_b):
    c = lax.axis_index("c")
    s = lax.axis_index("s")
    w = s * NC + c
    pltpu.sync_copy(av_g.at[w], idx_v)
    tables = (t0, t1, t2, t3)
    outs = (o0, o1, o2, o3)

    def start_all(j, rows, sem):
        for k, t in enumerate(tables):
            pltpu.make_async_copy(t.at[idx_v.at[j]], rows.at[k], sem).start()

    def drain_all(j, rows, sem):
        for k, t in enumerate(tables):
            pltpu.make_async_copy(t.at[idx_v.at[j]], rows.at[k], sem).wait()
        for k, o in enumerate(outs):
            pltpu.sync_copy(rows.at[k], o.at[pl.ds(w * APW + j * G, G)])

    start_all(0, ra, sem_a)

    def step(i, carry):
        start_all(2 * i + 1, rb, sem_b)
        drain_all(2 * i, ra, sem_a)
        start_all(2 * i + 2, ra, sem_a)
        drain_all(2 * i + 1, rb, sem_b)
        return carry

    lax.fori_loop(0, (AG - 1) // 2, step, 0)
    drain_all(AG - 1, ra, sem_a)


BM = 1000
BP = 512


def _bd4(w):
    z = jnp.zeros((D, D), jnp.float32)
    rows = [jnp.concatenate([w if j == k else z for j in range(4)], axis=1)
            for k in range(4)]
    return jnp.concatenate(rows, axis=0)


def _tc_mlp_body(x_ref, w1_ref, b1_ref, w2_ref, b2_ref, o_ref):
    h = jnp.dot(x_ref[...], w1_ref[0], preferred_element_type=jnp.float32)
    h = jnp.maximum(h + b1_ref[0], 0.0)
    o_ref[...] = jnp.dot(h, w2_ref[0], preferred_element_type=jnp.float32) + b2_ref[0]


def _tc_mlp(feats, w1s, b1s, w2s, b2s):
    grid = feats.shape[0] // BM
    half = grid // 2
    return pl.pallas_call(
        _tc_mlp_body,
        grid=(grid,),
        in_specs=[
            pl.BlockSpec((BM, 8), lambda i: (i, 0)),
            pl.BlockSpec((1, 8, D), lambda i: (i // half, 0, 0)),
            pl.BlockSpec((1, 1, D), lambda i: (i // half, 0, 0)),
            pl.BlockSpec((1, D, D), lambda i: (i // half, 0, 0)),
            pl.BlockSpec((1, 1, D), lambda i: (i // half, 0, 0)),
        ],
        out_specs=pl.BlockSpec((BM, D), lambda i: (i, 0)),
        out_shape=jax.ShapeDtypeStruct((feats.shape[0], D), jnp.float32),
    )(feats, w1s, b1s, w2s, b2s)


def _tc_combine_body(s0_ref, s1_ref, xp_ref, cp_ref, basis_ref, att_ref,
                     root_ref, bias_ref, o_ref):
    a0 = att_ref[0, 0]
    a1 = att_ref[1, 0]
    basis = basis_ref[...]
    aggr = jnp.dot(s0_ref[...], _bd4(a0 * basis),
                   preferred_element_type=jnp.float32)
    aggr += jnp.dot(s1_ref[...], _bd4(a1 * basis),
                    preferred_element_type=jnp.float32)
    tot = jnp.dot(cp_ref[...], _bd4(jnp.full((D, D), 1.0 / H, jnp.float32)),
                  preferred_element_type=jnp.float32)
    aggr = aggr / jnp.maximum(tot, 1.0)
    y = aggr + jnp.dot(xp_ref[...], _bd4(root_ref[...]),
                       preferred_element_type=jnp.float32)
    o_ref[...] = jnp.maximum(y + bias_ref[0], 0.0)


def _tc_combine(sp, xp, cp, basis, att, root, bias4):
    grid = XP // BP
    off = XP // BP
    return pl.pallas_call(
        _tc_combine_body,
        grid=(grid,),
        in_specs=[
            pl.BlockSpec((BP, 128), lambda i: (i, 0)),
            pl.BlockSpec((BP, 128), lambda i: (i + off, 0)),
            pl.BlockSpec((BP, 128), lambda i: (i, 0)),
            pl.BlockSpec((BP, 128), lambda i: (i, 0)),
            pl.BlockSpec((D, D), lambda i: (0, 0)),
            pl.BlockSpec(memory_space=pltpu.SMEM),
            pl.BlockSpec((D, D), lambda i: (0, 0)),
            pl.BlockSpec((1, 128), lambda i: (0, 0)),
        ],
        out_specs=pl.BlockSpec((BP, 128), lambda i: (i, 0)),
        out_shape=jax.ShapeDtypeStruct((XP, 128), jnp.float32),
    )(sp, sp, xp, cp, basis, att, root, bias4)


def _tc_final_body(g0, g1, g2, g3, w1_ref, b1_ref, w4_ref, b4_ref, o_ref):
    gs = (g0, g1, g2, g3)
    h = None
    for k, g in enumerate(gs):
        part = jnp.dot(g[...], _bd4(w1_ref[pl.ds(k * D, D)]),
                       preferred_element_type=jnp.float32)
        h = part if h is None else h + part
    h = jnp.maximum(h + b1_ref[0], 0.0)
    z = jnp.zeros((D, 1), jnp.float32)
    w4 = w4_ref[...]
    cols = [jnp.concatenate([w4 if j == k else z for j in range(4)], axis=0)
            for k in range(4)]
    f = jnp.concatenate(cols, axis=1)
    o_ref[...] = jnp.dot(h, f, preferred_element_type=jnp.float32) \
        + b4_ref[0, 0]


def _tc_final(gs, fc1_W, fc1_b4, fc4_W, fc4_b):
    grid = AP4 // BP
    return pl.pallas_call(
        _tc_final_body,
        grid=(grid,),
        in_specs=[pl.BlockSpec((BP, 128), lambda i: (i, 0)) for _ in range(4)]
        + [
            pl.BlockSpec((4 * D, D), lambda i: (0, 0)),
            pl.BlockSpec((1, 128), lambda i: (0, 0)),
            pl.BlockSpec((D, 1), lambda i: (0, 0)),
            pl.BlockSpec(memory_space=pltpu.SMEM),
        ],
        out_specs=pl.BlockSpec((BP, 4), lambda i: (i, 0)),
        out_shape=jax.ShapeDtypeStruct((AP4, 4), jnp.float32),
    )(*gs, fc1_W, fc1_b4, fc4_W, fc4_b.reshape(1, 1))


def kernel(var_node_features, con_node_features, node_types, assoc_var,
           assoc_con, edge_index, edge_types, edge_features, var_W1, var_b1,
           var_W2, var_b2, con_W1, con_b1, con_W2, con_b2, c1_basis, c1_att,
           c1_root, c1_bias, c2_basis, c2_att, c2_root, c2_bias, c3_basis,
           c3_att, c3_root, c3_bias, fc1_W, fc1_b, fc4_W, fc4_b):
    i32 = jnp.int32
    src = edge_index[0].astype(i32)
    dst = edge_index[1].astype(i32)
    et = edge_types.astype(i32)

    pad = EP - E
    zpad = jnp.zeros((pad,), i32)
    src_lo = jnp.concatenate([2 * src, zpad]).reshape(NT * LG, G)
    src_hi = jnp.concatenate([2 * src + 1, zpad + 1]).reshape(NT * LG, G)
    cidxp = jnp.concatenate([dst + N2 * et,
                             jnp.full((pad,), TRASH, i32)]).reshape(NT * LG, G)
    dstp = jnp.concatenate([dst, jnp.full((pad,), TRASH, i32)])
    dstp = dstp.reshape(NT * NC, CG, G)
    avp = jnp.concatenate(
        [assoc_var.astype(i32), jnp.zeros((AP - N // 2,), i32)]
    ).reshape(NT * NC, AG, G)

    zeros = jnp.zeros((SPT, H), jnp.float32)
    ones = jnp.ones((G, H), jnp.float32)

    feats = jnp.concatenate([var_node_features, con_node_features], axis=0)
    feats = jnp.pad(feats, ((0, 0), (0, 6)))
    w1s = jnp.stack([jnp.pad(var_W1, ((0, 6), (0, 0))),
                     jnp.pad(con_W1, ((0, 6), (0, 0)))])
    b1s = jnp.stack([var_b1.reshape(1, D), con_b1.reshape(1, D)])
    w2s = jnp.stack([var_W2, con_W2])
    b2s = jnp.stack([var_b2.reshape(1, D), con_b2.reshape(1, D)])
    y = _tc_mlp(feats, w1s, b1s, w2s, b2s)

    x0 = jnp.zeros((N2, D), jnp.float32)
    x0 = x0.at[jnp.concatenate([assoc_var, assoc_con])].set(y)
    xp0 = x0.reshape(XP, 128)

    cnt = _sc_count(dstp, zeros, ones)
    cp = cnt.reshape(XP, 128)

    xps = [xp0]
    for basis, att, root, bias in (
        (c1_basis, c1_att, c1_root, c1_bias),
        (c2_basis, c2_att, c2_root, c2_bias),
        (c3_basis, c3_att, c3_root, c3_bias),
    ):
        xp = xps[-1]
        s_out = _sc_layer(xp.reshape(2 * N2, H), src_lo, src_hi, cidxp, zeros)
        xps.append(_tc_combine(
            s_out.reshape(SP4, 128), xp, cp,
            basis.reshape(D, D), att.reshape(2, 1), root,
            jnp.tile(bias.reshape(1, D), (1, 4))))

    tables = [xp.reshape(N2, D) for xp in xps]
    gs = _sc_readout(*tables, avp)
    out = _tc_final([g.reshape(AP4, 128) for g in gs],
                    fc1_W, jnp.tile(fc1_b.reshape(1, D), (1, 4)),
                    fc4_W, fc4_b)
    return out.reshape(AP)[: N // 2]
